# baseline (speedup 1.0000x reference)
import functools

import jax
import jax.numpy as jnp
from jax import lax
from jax.experimental import pallas as pl
from jax.experimental.pallas import tpu as pltpu
from jax.experimental.pallas import tpu_sc as plsc

N_NODES = 10000
N_EDGES = 320000
D = 128
DH = D // 2
EPS = 1e-5

NC = 2
NS = 16
EPT = N_EDGES // NS
CHUNK = 128
NCHUNK = EPT // CHUNK
TAIL = EPT - NCHUNK * CHUNK
RING = 4
ZCH = 80
NODE_CHUNKS = N_NODES // ZCH


def _sc_segment_sum(h2, src, dst):
    mesh = plsc.VectorSubcoreMesh(core_axis_name="c", subcore_axis_name="s")

    @functools.partial(
        pl.kernel,
        out_type=jax.ShapeDtypeStruct((N_NODES, D), jnp.float32),
        mesh=mesh,
        compiler_params=pltpu.CompilerParams(use_tc_tiling_on_sc=False),
        scratch_types=[
            pltpu.VMEM((EPT,), jnp.int32),
            pltpu.VMEM((EPT,), jnp.int32),
            [pltpu.VMEM((CHUNK,), jnp.int32) for _ in range(RING)],
            [pltpu.VMEM((CHUNK,), jnp.int32) for _ in range(RING)],
            [pltpu.VMEM((CHUNK, DH), jnp.float32) for _ in range(RING)],
            pltpu.VMEM((TAIL,), jnp.int32),
            pltpu.VMEM((TAIL,), jnp.int32),
            pltpu.VMEM((TAIL, DH), jnp.float32),
            pltpu.VMEM((ZCH, DH), jnp.float32),
            pltpu.VMEM_SHARED((N_NODES, DH), jnp.float32),
            [pltpu.SemaphoreType.DMA for _ in range(RING)],
            [pltpu.SemaphoreType.DMA for _ in range(RING)],
        ],
    )
    def k(h_hbm, src_hbm, dst_hbm, out_hbm,
          src_v, dst_v, sidx, didx, rows, sidx_t, didx_t, rows_t,
          zbuf, acc, gsem, ssem):
        c = lax.axis_index("c")
        s = lax.axis_index("s")

        pltpu.sync_copy(src_hbm.at[pl.ds(s * EPT, EPT)], src_v)
        pltpu.sync_copy(dst_hbm.at[pl.ds(s * EPT, EPT)], dst_v)

        def zstore(i, carry):
            zbuf[i // (DH // 16), pl.ds((i % (DH // 16)) * 16, 16)] = (
                jnp.zeros((16,), jnp.float32))
            return carry
        lax.fori_loop(0, ZCH * (DH // 16), zstore, 0)

        def strided_node_chunks(body):
            def it(i, carry):
                cid = s + i * NS

                @pl.when(cid < NODE_CHUNKS)
                def _():
                    body(cid)
                return carry
            lax.fori_loop(0, (NODE_CHUNKS + NS - 1) // NS, it, 0)

        strided_node_chunks(
            lambda cid: pltpu.sync_copy(zbuf, acc.at[pl.ds(cid * ZCH, ZCH)]))
        plsc.subcore_barrier()

        def prep_into(sbuf, dbuf, j, n):
            for q in range(n // 16):
                e0 = j * CHUNK + q * 16
                sbuf[pl.ds(q * 16, 16)] = src_v[pl.ds(e0, 16)] * 2 + c
                dbuf[pl.ds(q * 16, 16)] = dst_v[pl.ds(e0, 16)]

        def prep_idx(b, j):
            prep_into(sidx[b], didx[b], j, CHUNK)

        for b in range(RING):
            prep_idx(b, b)
            pltpu.async_copy(h_hbm.at[sidx[b]], rows[b], gsem[b])

        def block(jb, carry):
            for b in range(RING):
                pltpu.make_async_copy(
                    h_hbm.at[sidx[b]], rows[b], gsem[b]).wait()
                pltpu.async_copy(
                    rows[b], acc.at[didx[b]], ssem[b], add=True)
            for b in range(RING):
                j = jb * RING + b
                pltpu.make_async_copy(
                    rows[b], acc.at[didx[b]], ssem[b]).wait()
                jn = j + RING

                @pl.when(jn < NCHUNK)
                def _():
                    prep_idx(b, jn)
                    pltpu.async_copy(h_hbm.at[sidx[b]], rows[b], gsem[b])
            return carry
        lax.fori_loop(0, NCHUNK // RING, block, 0)

        prep_into(sidx_t, didx_t, NCHUNK, TAIL)
        pltpu.async_copy(h_hbm.at[sidx_t], rows_t, gsem[0]).wait()
        pltpu.sync_copy(rows_t, acc.at[didx_t], add=True)
        plsc.subcore_barrier()

        strided_node_chunks(
            lambda cid: pltpu.sync_copy(
                acc.at[pl.ds(cid * ZCH, ZCH)],
                out_hbm.at[pl.ds(cid * ZCH, ZCH), pl.ds(c * DH, DH)]))

    return k(h2, src, dst)


def _bn_body(agg_ref, gamma_ref, beta_ref, out_ref):
    agg = agg_ref[...]
    mean = jnp.mean(agg, axis=0, keepdims=True)
    cent = agg - mean
    var = jnp.mean(cent * cent, axis=0, keepdims=True)
    out_ref[...] = cent * lax.rsqrt(var + EPS) * gamma_ref[...] + beta_ref[...]


def kernel(h, edge_index, gamma, beta):
    h2 = h.reshape(2 * N_NODES, DH)
    agg = _sc_segment_sum(h2, edge_index[0], edge_index[1])
    return pl.pallas_call(
        _bn_body,
        out_shape=jax.ShapeDtypeStruct((N_NODES, D), jnp.float32),
    )(agg, gamma.reshape(1, D), beta.reshape(1, D))
